# baseline (speedup 1.0000x reference)
import functools

import numpy as np
import jax
import jax.numpy as jnp
from jax import lax
from jax.experimental import pallas as pl
from jax.experimental.pallas import tpu as pltpu
from jax.experimental.pallas import tpu_sc as plsc

_NUM_T = 1000
_B = 64
_C = 3
_N = 2048
_L = 16
_NW = 32
_RG = 8
_CW = 512
_ROWS = _B * _C


def _make_coef_table() -> np.ndarray:
    betas = np.linspace(0.0001, 0.02, _NUM_T).astype(np.float64)
    alphas = 1.0 - betas
    acp = np.cumprod(alphas, axis=0)
    acp_prev = np.append(1.0, acp[:-1])
    sqrt_recip = np.sqrt(1.0 / acp)
    sqrt_recipm1 = np.sqrt(1.0 / acp - 1.0)
    post_var = betas * (1.0 - acp_prev) / (1.0 - acp)
    log_var = np.log(np.maximum(post_var, 1e-20)).astype(np.float32)
    sig = np.exp(0.5 * log_var.astype(np.float64)).astype(np.float32)
    sig[0] = 0.0
    coef1 = betas * np.sqrt(acp_prev) / (1.0 - acp)
    coef2 = (1.0 - acp_prev) * np.sqrt(alphas) / (1.0 - acp)
    tab = np.stack([
        sqrt_recip.astype(np.float32),
        sqrt_recipm1.astype(np.float32),
        coef1.astype(np.float32),
        coef2.astype(np.float32),
        sig,
    ], axis=1)
    tab = np.repeat(tab, _L, axis=1).astype(np.float32)
    return np.pad(tab, ((0, 0), (0, 128 - 5 * _L)))


_TAB = _make_coef_table()


def _sc_body(tab_hbm, t_hbm, d_hbm, m_hbm, n_hbm, samp_hbm, xr_hbm,
             idx_v, rows_v, d_v, m_v, n_v, s_v, xr_v,
             sem_g, sems_d, sems_m, sems_n, sems_o1, sems_o2):
    wid = lax.axis_index("s") * 2 + lax.axis_index("c")
    rg = wid % 8
    ch = wid // 8
    col = ch * _CW

    pltpu.sync_copy(t_hbm.at[pl.ds(rg * _RG, _RG)], idx_v)
    cg = pltpu.async_copy(tab_hbm.at[idx_v], rows_v, sem_g)

    hd, hm, hn = [], [], []
    for c in range(_C):
        rows = pl.ds(c * _B + rg * _RG, _RG)
        dst = pl.ds(c * _RG, _RG)
        hd.append(pltpu.async_copy(
            d_hbm.at[rows, pl.ds(col, _CW)], d_v.at[dst], sems_d.at[c]))
        hm.append(pltpu.async_copy(
            m_hbm.at[rows, pl.ds(col, _CW)], m_v.at[dst], sems_m.at[c]))
        hn.append(pltpu.async_copy(
            n_hbm.at[rows, pl.ds(col, _CW)], n_v.at[dst], sems_n.at[c]))
    cg.wait()

    o1, o2 = [], []
    for c in range(_C):
        hd[c].wait()
        hm[c].wait()
        hn[c].wait()
        for s in range(_RG):
            row = c * _RG + s
            ca = rows_v[s, pl.ds(0 * _L, _L)]
            cb = rows_v[s, pl.ds(1 * _L, _L)]
            c1 = rows_v[s, pl.ds(2 * _L, _L)]
            c2 = rows_v[s, pl.ds(3 * _L, _L)]
            cs = rows_v[s, pl.ds(4 * _L, _L)]

            @plsc.parallel_loop(0, _CW // _L, unroll=4)
            def _step(i, row=row, ca=ca, cb=cb, c1=c1, c2=c2, cs=cs):
                off = i * _L
                d = d_v[row, pl.ds(off, _L)]
                m = m_v[row, pl.ds(off, _L)]
                n = n_v[row, pl.ds(off, _L)]
                xr = jnp.clip(ca * d - cb * m, -0.5, 0.5)
                xr_v[row, pl.ds(off, _L)] = xr
                s_v[row, pl.ds(off, _L)] = c1 * xr + c2 * d + cs * n

        rows = pl.ds(c * _B + rg * _RG, _RG)
        src = pl.ds(c * _RG, _RG)
        o1.append(pltpu.async_copy(
            s_v.at[src], samp_hbm.at[rows, pl.ds(col, _CW)], sems_o1.at[c]))
        o2.append(pltpu.async_copy(
            xr_v.at[src], xr_hbm.at[rows, pl.ds(col, _CW)], sems_o2.at[c]))
    for c in range(_C):
        o1[c].wait()
        o2[c].wait()


_sc_call = functools.partial(
    pl.kernel,
    mesh=plsc.VectorSubcoreMesh(core_axis_name="c", subcore_axis_name="s"),
    out_type=(
        jax.ShapeDtypeStruct((_ROWS, _N), jnp.float32),
        jax.ShapeDtypeStruct((_ROWS, _N), jnp.float32),
    ),
    scratch_types=[
        pltpu.VMEM((_RG,), jnp.int32),
        pltpu.VMEM((_RG, 128), jnp.float32),
        pltpu.VMEM((_C * _RG, _CW), jnp.float32),
        pltpu.VMEM((_C * _RG, _CW), jnp.float32),
        pltpu.VMEM((_C * _RG, _CW), jnp.float32),
        pltpu.VMEM((_C * _RG, _CW), jnp.float32),
        pltpu.VMEM((_C * _RG, _CW), jnp.float32),
        pltpu.SemaphoreType.DMA,
        pltpu.SemaphoreType.DMA((_C,)),
        pltpu.SemaphoreType.DMA((_C,)),
        pltpu.SemaphoreType.DMA((_C,)),
        pltpu.SemaphoreType.DMA((_C,)),
        pltpu.SemaphoreType.DMA((_C,)),
    ],
    compiler_params=pltpu.CompilerParams(use_tc_tiling_on_sc=True),
)(_sc_body)


def kernel(data, t, model_output, noise):
    b, c, n = data.shape
    dt = jnp.transpose(data, (1, 0, 2)).reshape(_ROWS, _N)
    mt = jnp.transpose(model_output, (1, 0, 2)).reshape(_ROWS, _N)
    nt = jnp.transpose(noise, (1, 0, 2)).reshape(_ROWS, _N)
    samp, xr = _sc_call(_TAB, t.astype(jnp.int32), dt, mt, nt)
    samp = jnp.transpose(samp.reshape(c, b, n), (1, 0, 2))
    xr = jnp.transpose(xr.reshape(c, b, n), (1, 0, 2))
    return samp, xr


# speedup vs baseline: 1.0046x; 1.0046x over previous
import functools

import numpy as np
import jax
import jax.numpy as jnp
from jax import lax
from jax.experimental import pallas as pl
from jax.experimental.pallas import tpu as pltpu
from jax.experimental.pallas import tpu_sc as plsc

_NUM_T = 1000
_B = 64
_C = 3
_N = 2048
_L = 16
_NW = 32
_RG = 8
_CW = 512
_ROWS = _B * _C


def _make_coef_table() -> np.ndarray:
    betas = np.linspace(0.0001, 0.02, _NUM_T).astype(np.float64)
    alphas = 1.0 - betas
    acp = np.cumprod(alphas, axis=0)
    acp_prev = np.append(1.0, acp[:-1])
    sqrt_recip = np.sqrt(1.0 / acp)
    sqrt_recipm1 = np.sqrt(1.0 / acp - 1.0)
    post_var = betas * (1.0 - acp_prev) / (1.0 - acp)
    log_var = np.log(np.maximum(post_var, 1e-20)).astype(np.float32)
    sig = np.exp(0.5 * log_var.astype(np.float64)).astype(np.float32)
    sig[0] = 0.0
    coef1 = betas * np.sqrt(acp_prev) / (1.0 - acp)
    coef2 = (1.0 - acp_prev) * np.sqrt(alphas) / (1.0 - acp)
    tab = np.stack([
        sqrt_recip.astype(np.float32),
        sqrt_recipm1.astype(np.float32),
        coef1.astype(np.float32),
        coef2.astype(np.float32),
        sig,
    ], axis=1)
    tab = np.repeat(tab, _L, axis=1).astype(np.float32)
    return np.pad(tab, ((0, 0), (0, 128 - 5 * _L)))


_TAB = _make_coef_table()


def _sc_body(tab_hbm, t_hbm, d_hbm, m_hbm, n_hbm, samp_hbm, xr_hbm,
             idx_v, rows_v, d_v, m_v, n_v, s_v, xr_v,
             sem_g, sems_d, sems_m, sems_n, sems_o1, sems_o2):
    wid = lax.axis_index("s") * 2 + lax.axis_index("c")
    rg = wid % 8
    ch = wid // 8
    col = ch * _CW

    pltpu.sync_copy(t_hbm.at[pl.ds(rg * _RG, _RG)], idx_v)
    cg = pltpu.async_copy(tab_hbm.at[idx_v], rows_v, sem_g)

    hd, hm, hn = [], [], []
    for c in range(_C):
        rows = pl.ds(c * _B + rg * _RG, _RG)
        dst = pl.ds(c * _RG, _RG)
        hd.append(pltpu.async_copy(
            d_hbm.at[rows, pl.ds(col, _CW)], d_v.at[dst], sems_d.at[c]))
        hm.append(pltpu.async_copy(
            m_hbm.at[rows, pl.ds(col, _CW)], m_v.at[dst], sems_m.at[c]))
        hn.append(pltpu.async_copy(
            n_hbm.at[rows, pl.ds(col, _CW)], n_v.at[dst], sems_n.at[c]))
    cg.wait()

    o1, o2 = [], []
    for c in range(_C):
        hd[c].wait()
        hm[c].wait()
        hn[c].wait()
        for s in range(_RG):
            row = c * _RG + s
            ca = rows_v[s, pl.ds(0 * _L, _L)]
            cb = rows_v[s, pl.ds(1 * _L, _L)]
            c1 = rows_v[s, pl.ds(2 * _L, _L)]
            c2 = rows_v[s, pl.ds(3 * _L, _L)]
            cs = rows_v[s, pl.ds(4 * _L, _L)]

            @plsc.parallel_loop(0, _CW // _L, unroll=4)
            def _step(i, row=row, ca=ca, cb=cb, c1=c1, c2=c2, cs=cs):
                off = i * _L
                d = d_v[row, pl.ds(off, _L)]
                m = m_v[row, pl.ds(off, _L)]
                n = n_v[row, pl.ds(off, _L)]
                xr = jnp.clip(ca * d - cb * m, -0.5, 0.5)
                xr_v[row, pl.ds(off, _L)] = xr
                s_v[row, pl.ds(off, _L)] = c1 * xr + c2 * d + cs * n

        rows = pl.ds(c * _B + rg * _RG, _RG)
        src = pl.ds(c * _RG, _RG)
        o1.append(pltpu.async_copy(
            s_v.at[src], samp_hbm.at[rows, pl.ds(col, _CW)], sems_o1.at[c]))
        o2.append(pltpu.async_copy(
            xr_v.at[src], xr_hbm.at[rows, pl.ds(col, _CW)], sems_o2.at[c]))
    for c in range(_C):
        o1[c].wait()
        o2[c].wait()


_sc_call = functools.partial(
    pl.kernel,
    mesh=plsc.VectorSubcoreMesh(core_axis_name="c", subcore_axis_name="s"),
    out_type=(
        jax.ShapeDtypeStruct((_ROWS, _N), jnp.float32),
        jax.ShapeDtypeStruct((_ROWS, _N), jnp.float32),
    ),
    scratch_types=[
        pltpu.VMEM((_RG,), jnp.int32),
        pltpu.VMEM((_RG, 128), jnp.float32),
        pltpu.VMEM((_C * _RG, _CW), jnp.float32),
        pltpu.VMEM((_C * _RG, _CW), jnp.float32),
        pltpu.VMEM((_C * _RG, _CW), jnp.float32),
        pltpu.VMEM((_C * _RG, _CW), jnp.float32),
        pltpu.VMEM((_C * _RG, _CW), jnp.float32),
        pltpu.SemaphoreType.DMA,
        pltpu.SemaphoreType.DMA((_C,)),
        pltpu.SemaphoreType.DMA((_C,)),
        pltpu.SemaphoreType.DMA((_C,)),
        pltpu.SemaphoreType.DMA((_C,)),
        pltpu.SemaphoreType.DMA((_C,)),
    ],
    compiler_params=pltpu.CompilerParams(
        use_tc_tiling_on_sc=True, skip_device_barrier=True),
)(_sc_body)


def kernel(data, t, model_output, noise):
    b, c, n = data.shape
    dt = jnp.transpose(data, (1, 0, 2)).reshape(_ROWS, _N)
    mt = jnp.transpose(model_output, (1, 0, 2)).reshape(_ROWS, _N)
    nt = jnp.transpose(noise, (1, 0, 2)).reshape(_ROWS, _N)
    samp, xr = _sc_call(_TAB, t.astype(jnp.int32), dt, mt, nt)
    samp = jnp.transpose(samp.reshape(c, b, n), (1, 0, 2))
    xr = jnp.transpose(xr.reshape(c, b, n), (1, 0, 2))
    return samp, xr
